# Initial kernel scaffold; baseline (speedup 1.0000x reference)
import jax
import jax.numpy as jnp
from jax.experimental import pallas as pl


def kernel(x, idx):
    raise NotImplementedError("write your pallas kernel here")


# speedup vs baseline: 1.5410x; 1.5410x over previous
import dataclasses

import jax
import jax.numpy as jnp
from jax import lax
from jax.experimental import pallas as pl
from jax.experimental.pallas import tpu as pltpu
from jax.experimental.pallas import tpu_sc as plsc

M = 1536
N = 50176
NW = 32
ROWS_PER_W = M // NW
CHUNK = 7168
NCHUNK = N // CHUNK
VEC = 16
UNROLL = 4


def _jumble_body(x_hbm, idx_hbm, out_hbm, idx_v, row_v, out_v):
    wid = lax.axis_index("s") * 2 + lax.axis_index("c")
    pltpu.sync_copy(idx_hbm, idx_v)
    base = wid * ROWS_PER_W

    @pl.loop(0, ROWS_PER_W)
    def _row(r):
        row = base + r
        pltpu.sync_copy(x_hbm.at[row], row_v)

        @pl.loop(0, NCHUNK)
        def _chunk(k):
            off = k * CHUNK

            @pl.loop(0, CHUNK, step=VEC * UNROLL)
            def _vec(i):
                for u in range(UNROLL):
                    o = i + u * VEC
                    iv = idx_v[pl.ds(off + o, VEC)]
                    out_v[pl.ds(o, VEC)] = plsc.load_gather(row_v, [iv])

            pltpu.sync_copy(out_v, out_hbm.at[row, pl.ds(off, CHUNK)])


def kernel(x, idx):
    b, c, h, w = x.shape
    xf = x.reshape(M, N)
    idx32 = idx.astype(jnp.int32)
    mesh = plsc.VectorSubcoreMesh(core_axis_name="c", subcore_axis_name="s")
    cp = pltpu.CompilerParams()
    if "needs_layout_passes" in pltpu.CompilerParams.__dataclass_fields__:
        cp = dataclasses.replace(cp, needs_layout_passes=False)
    run = pl.kernel(
        _jumble_body,
        out_type=jax.ShapeDtypeStruct((M, N), jnp.float32),
        mesh=mesh,
        scratch_types=[
            pltpu.VMEM((N,), jnp.int32),
            pltpu.VMEM((N,), jnp.float32),
            pltpu.VMEM((CHUNK,), jnp.float32),
        ],
        compiler_params=cp,
    )
    return run(xf, idx32).reshape(b, c, h, w)


# speedup vs baseline: 3.2294x; 2.0957x over previous
import dataclasses

import jax
import jax.numpy as jnp
from jax import lax
from jax.experimental import pallas as pl
from jax.experimental.pallas import tpu as pltpu
from jax.experimental.pallas import tpu_sc as plsc

M = 1536
N = 50176
NW = 32
ROWS_PER_W = M // NW
CHUNK = 7168
NCHUNK = N // CHUNK
VEC = 16
UNROLL = 4


def _jumble_body(x_hbm, idx_hbm, out_hbm, idx_v, row_v, out_v, sem):
    wid = lax.axis_index("s") * 2 + lax.axis_index("c")
    pltpu.sync_copy(idx_hbm, idx_v)
    base = wid * ROWS_PER_W

    @pl.loop(0, ROWS_PER_W)
    def _row(r):
        row = base + r
        pltpu.sync_copy(x_hbm.at[row], row_v)

        copies = []
        for k in range(NCHUNK):
            slot = k % 2
            if k >= 2:
                copies[k - 2].wait()

            @plsc.parallel_loop(0, CHUNK, step=VEC, unroll=UNROLL)
            def _vec(i, off=k * CHUNK, slot=slot):
                iv = idx_v[pl.ds(off + i, VEC)]
                out_v[slot, pl.ds(i, VEC)] = plsc.load_gather(row_v, [iv])

            cp = pltpu.make_async_copy(
                out_v.at[slot],
                out_hbm.at[row, pl.ds(k * CHUNK, CHUNK)],
                sem.at[slot],
            )
            cp.start()
            copies.append(cp)
        copies[-2].wait()
        copies[-1].wait()


def kernel(x, idx):
    b, c, h, w = x.shape
    xf = x.reshape(M, N)
    idx32 = idx.astype(jnp.int32)
    mesh = plsc.VectorSubcoreMesh(core_axis_name="c", subcore_axis_name="s")
    cp = pltpu.CompilerParams()
    if "needs_layout_passes" in pltpu.CompilerParams.__dataclass_fields__:
        cp = dataclasses.replace(cp, needs_layout_passes=False)
    run = pl.kernel(
        _jumble_body,
        out_type=jax.ShapeDtypeStruct((M, N), jnp.float32),
        mesh=mesh,
        scratch_types=[
            pltpu.VMEM((N,), jnp.int32),
            pltpu.VMEM((N,), jnp.float32),
            pltpu.VMEM((2, CHUNK), jnp.float32),
            pltpu.SemaphoreType.DMA((2,)),
        ],
        compiler_params=cp,
    )
    return run(xf, idx32).reshape(b, c, h, w)


# speedup vs baseline: 3.2887x; 1.0184x over previous
import dataclasses

import jax
import jax.numpy as jnp
from jax import lax
from jax.experimental import pallas as pl
from jax.experimental.pallas import tpu as pltpu
from jax.experimental.pallas import tpu_sc as plsc

M = 1536
N = 50176
NW = 32
ROWS_PER_W = M // NW
CHUNK = 7168
NCHUNK = N // CHUNK
VEC = 16
UNROLL = 8


def _jumble_body(x_hbm, idx_hbm, out_hbm, idx_v, row_v, out_v, sem):
    wid = lax.axis_index("s") * 2 + lax.axis_index("c")
    pltpu.sync_copy(idx_hbm, idx_v)
    base = wid * ROWS_PER_W

    @pl.loop(0, ROWS_PER_W)
    def _row(r):
        row = base + r
        pltpu.sync_copy(x_hbm.at[row], row_v)

        copies = []
        for k in range(NCHUNK):
            slot = k % 2
            if k >= 2:
                copies[k - 2].wait()

            @plsc.parallel_loop(0, CHUNK, step=VEC, unroll=UNROLL)
            def _vec(i, off=k * CHUNK, slot=slot):
                iv = idx_v[pl.ds(off + i, VEC)]
                out_v[slot, pl.ds(i, VEC)] = plsc.load_gather(row_v, [iv])

            cp = pltpu.make_async_copy(
                out_v.at[slot],
                out_hbm.at[row, pl.ds(k * CHUNK, CHUNK)],
                sem.at[slot],
            )
            cp.start()
            copies.append(cp)
        copies[-2].wait()
        copies[-1].wait()


def kernel(x, idx):
    b, c, h, w = x.shape
    xf = x.reshape(M, N)
    idx32 = idx.astype(jnp.int32)
    mesh = plsc.VectorSubcoreMesh(core_axis_name="c", subcore_axis_name="s")
    cp = pltpu.CompilerParams()
    if "needs_layout_passes" in pltpu.CompilerParams.__dataclass_fields__:
        cp = dataclasses.replace(cp, needs_layout_passes=False)
    run = pl.kernel(
        _jumble_body,
        out_type=jax.ShapeDtypeStruct((M, N), jnp.float32),
        mesh=mesh,
        scratch_types=[
            pltpu.VMEM((N,), jnp.int32),
            pltpu.VMEM((N,), jnp.float32),
            pltpu.VMEM((2, CHUNK), jnp.float32),
            pltpu.SemaphoreType.DMA((2,)),
        ],
        compiler_params=cp,
    )
    return run(xf, idx32).reshape(b, c, h, w)


# speedup vs baseline: 5.9966x; 1.8234x over previous
import dataclasses

import jax
import jax.numpy as jnp
from jax import lax
from jax.experimental import pallas as pl
from jax.experimental.pallas import tpu as pltpu
from jax.experimental.pallas import tpu_sc as plsc

M = 1536
H = 224
W = 224
N = H * W
NW = 32
ROWS_PER_W = M // NW
HCHUNK = 32
NCHUNK = H // HCHUNK
VEC = 16
WVECS = W // VEC


def _jumble_body(x_hbm, code_hbm, out_hbm, code_v, row_v, out_v, sem):
    wid = lax.axis_index("s") * 2 + lax.axis_index("c")
    pltpu.sync_copy(code_hbm, code_v)
    base = wid * ROWS_PER_W

    @pl.loop(0, ROWS_PER_W)
    def _row(r):
        row = base + r
        pltpu.sync_copy(x_hbm.at[row], row_v)

        copies = []
        for k in range(NCHUNK):
            slot = k % 2
            if k >= 2:
                copies[k - 2].wait()

            @pl.loop(0, HCHUNK)
            def _h(h, k=k, slot=slot):
                off = (k * HCHUNK + h) * W

                @plsc.parallel_loop(0, W, step=VEC, unroll=WVECS)
                def _vec(i, h=h, off=off, slot=slot):
                    c = code_v[pl.ds(off + i, VEC)]
                    ih = jnp.right_shift(c, 16)
                    iw = jnp.bitwise_and(c, 0xFFFF)
                    out_v[slot, h, pl.ds(i, VEC)] = plsc.load_gather(
                        row_v, [ih, iw])

            cp = pltpu.make_async_copy(
                out_v.at[slot],
                out_hbm.at[row, pl.ds(k * HCHUNK, HCHUNK), :],
                sem.at[slot],
            )
            cp.start()
            copies.append(cp)
        copies[-2].wait()
        copies[-1].wait()


def kernel(x, idx):
    b, c, h, w = x.shape
    x3 = x.reshape(M, H, W)
    idx32 = idx.astype(jnp.int32)
    code = jnp.left_shift(idx32 // W, 16) | (idx32 % W)
    mesh = plsc.VectorSubcoreMesh(core_axis_name="c", subcore_axis_name="s")
    cp = pltpu.CompilerParams()
    if "needs_layout_passes" in pltpu.CompilerParams.__dataclass_fields__:
        cp = dataclasses.replace(cp, needs_layout_passes=False)
    run = pl.kernel(
        _jumble_body,
        out_type=jax.ShapeDtypeStruct((M, H, W), jnp.float32),
        mesh=mesh,
        scratch_types=[
            pltpu.VMEM((N,), jnp.int32),
            pltpu.VMEM((H, W), jnp.float32),
            pltpu.VMEM((2, HCHUNK, W), jnp.float32),
            pltpu.SemaphoreType.DMA((2,)),
        ],
        compiler_params=cp,
    )
    return run(x3, code).reshape(b, c, h, w)
